# baseline (speedup 1.0000x reference)
import functools

import jax
import jax.numpy as jnp
from jax import lax
from jax.experimental import pallas as pl
from jax.experimental.pallas import tpu as pltpu
from jax.experimental.pallas import tpu_sc as plsc

HID = 128
NC = 250
NP = 5000
E = 512000
L = 200000

STRIPE = 80000
NPAD = 16 * STRIPE
ZCHUNK = 8000
EPW = E // 32
LPW = 6272
LPAD = 32 * LPW
GR = LPW // 128
AHALF = NP * 128
SCOLS = 5120
ENC_ROWS = 1000
SROWS = 5120
SHALF = SROWS * 128

def _sc_mesh():
    return plsc.VectorSubcoreMesh(
        core_axis_name="c", subcore_axis_name="s", num_cores=2,
        num_subcores=16)


def _hist_body(esrc, edst, zeros, out, acc, src_v, dst_v, idx_v, ones_v, sem):
    c = lax.axis_index("c")
    s = lax.axis_index("s")
    stripe = pl.ds(s * STRIPE, STRIPE)

    one16 = jnp.ones((16,), jnp.float32)

    def fill_ones(i, _):
        ones_v[pl.ds(i * 16, 16)] = one16
        return 0
    lax.fori_loop(0, 128 // 16, fill_ones, 0)

    pltpu.sync_copy(zeros.at[stripe], acc.at[stripe])

    base = (c * 16 + s) * EPW
    pltpu.sync_copy(esrc.at[pl.ds(base, EPW)], src_v)
    pltpu.sync_copy(edst.at[pl.ds(base, EPW)], dst_v)
    plsc.subcore_barrier()

    def row(j, _):
        for k in range(8):
            off = j * 128 + k * 16
            sv = src_v[pl.ds(off, 16)]
            dv = dst_v[pl.ds(off, 16)]
            hbase = (sv >> 7) * (AHALF - 128)
            idx_v[j, pl.ds(k * 16, 16)] = hbase + dv * 128 + sv
        pltpu.async_copy(ones_v, acc.at[idx_v.at[j]], sem, add=True)
        return 0
    lax.fori_loop(0, EPW // 128, row, 0)

    def drain(j, _):
        pltpu.make_async_copy(ones_v, acc.at[idx_v.at[0]], sem).wait()
        return 0
    lax.fori_loop(0, EPW // 128, drain, 0)
    plsc.subcore_barrier()

    pltpu.sync_copy(acc.at[stripe],
                    out.at[pl.ds(c * NPAD + s * STRIPE, STRIPE)])


@functools.cache
def _edge_histogram():
    return pl.kernel(
        _hist_body,
        out_type=jax.ShapeDtypeStruct((2 * NPAD,), jnp.float32),
        mesh=_sc_mesh(),
        scratch_types=[
            pltpu.VMEM_SHARED((NPAD,), jnp.float32),
            pltpu.VMEM((EPW,), jnp.int32),
            pltpu.VMEM((EPW,), jnp.int32),
            pltpu.VMEM((EPW // 128, 128), jnp.int32),
            pltpu.VMEM((128,), jnp.float32),
            pltpu.SemaphoreType.DMA,
        ],
    )


GROWS = 56


def _gather_body(ls, ld, stab, out, sv_ref, dv_ref, idx_v, out_v, sem):
    c = lax.axis_index("c")
    s = lax.axis_index("s")
    w = c * 16 + s
    base = w * LPW
    pltpu.sync_copy(ls.at[pl.ds(base, LPW)], sv_ref)
    pltpu.sync_copy(ld.at[pl.ds(base, LPW)], dv_ref)

    def row(j, _):
        for k in range(8):
            off = j * 128 + k * 16
            a = sv_ref[pl.ds(off, 16)]
            b = dv_ref[pl.ds(off, 16)]
            gbase = (a >> 7) * (SHALF - 128)
            idx_v[j, pl.ds(k * 16, 16)] = gbase + b * 128 + a
        pltpu.async_copy(stab.at[idx_v.at[j]], out_v.at[j], sem)
        return 0
    lax.fori_loop(0, GR, row, 0)

    def drain(j, _):
        pltpu.make_async_copy(stab.at[idx_v.at[0]], out_v.at[0], sem).wait()
        return 0
    lax.fori_loop(0, GR, drain, 0)
    pltpu.sync_copy(out_v, out.at[pl.ds(w * GROWS, GROWS)])


@functools.cache
def _label_gather():
    return pl.kernel(
        _gather_body,
        out_type=jax.ShapeDtypeStruct((32 * GROWS, 128), jnp.float32),
        mesh=_sc_mesh(),
        scratch_types=[
            pltpu.VMEM((LPW,), jnp.int32),
            pltpu.VMEM((LPW,), jnp.int32),
            pltpu.VMEM((GR, 128), jnp.int32),
            pltpu.VMEM((GROWS, 128), jnp.float32),
            pltpu.SemaphoreType.DMA,
        ],
    )


def _enc_body(x_ref, w_ref, b_ref, e_ref, o_ref):
    o_ref[...] = (
        jnp.dot(x_ref[...], w_ref[...], preferred_element_type=jnp.float32)
        + b_ref[...] + e_ref[...])


def _dense_body(ae0_ref, ao0_ref, ae1_ref, ao1_ref, xc_ref, wc_ref, bc_ref,
                ec_ref, hp_ref,
                w1rl_ref, b1rl_ref, w1rr_ref, w1fl_ref, b1fl_ref, w1fr_ref,
                w2rl_ref, b2rl_ref, w2rr_ref, w2fl_ref, b2fl_ref, w2fr_ref,
                s_ref):
    f32 = jnp.float32

    def dot(p, q):
        return lax.dot_general(p, q, (((1,), (0,)), ((), ())),
                               preferred_element_type=f32)

    def dot0(p, q):
        return lax.dot_general(p, q, (((0,), (0,)), ((), ())),
                               preferred_element_type=f32)

    ae = ae0_ref[...] + ae1_ref[...]
    ao = ao0_ref[...] + ao1_ref[...]
    rinv = 1.0 / jnp.maximum(
        jnp.sum(ae, axis=1, keepdims=True) + jnp.sum(ao, axis=1, keepdims=True),
        1.0)
    an_e = ae * rinv
    an_o = ao * rinv
    atn_e = ae / jnp.maximum(jnp.sum(ae, axis=0, keepdims=True), 1.0)
    atn_o = ao / jnp.maximum(jnp.sum(ao, axis=0, keepdims=True), 1.0)

    h_c = dot(xc_ref[...], wc_ref[...]) + bc_ref[...] + ec_ref[...]
    h_p = hp_ref[...]
    zpad = jnp.zeros((256 - NC, HID), f32)

    def mean_p(xc256):
        return dot(an_e, xc256[:128]) + dot(an_o, xc256[128:])

    def mean_c(xp):
        return jnp.concatenate([dot0(atn_e, xp), dot0(atn_o, xp)], axis=0)[:NC]

    hcp = jnp.concatenate([h_c, zpad], axis=0)
    h_c1 = jnp.maximum(
        dot(mean_c(h_p), w1rl_ref[...]) + b1rl_ref[...]
        + dot(h_c, w1rr_ref[...]), 0.0)
    h_p1 = jnp.maximum(
        dot(mean_p(hcp), w1fl_ref[...]) + b1fl_ref[...]
        + dot(h_p, w1fr_ref[...]), 0.0)

    hc1p = jnp.concatenate([h_c1, zpad], axis=0)
    h_c2 = (dot(mean_c(h_p1), w2rl_ref[...]) + b2rl_ref[...]
            + dot(h_c1, w2rr_ref[...]))
    h_p2 = (dot(mean_p(hc1p), w2fl_ref[...]) + b2fl_ref[...]
            + dot(h_p1, w2fr_ref[...]))

    hc2p = jnp.concatenate([h_c2, zpad], axis=0)
    oe = lax.dot_general(h_p2, hc2p[:128], (((1,), (1,)), ((), ())),
                         preferred_element_type=f32)
    oo = lax.dot_general(h_p2, hc2p[128:], (((1,), (1,)), ((), ())),
                         preferred_element_type=f32)
    spad = jnp.zeros((SROWS - NP, 128), f32)
    s_ref[...] = jnp.concatenate([oe, spad, oo, spad], axis=0)


def _dense_stack(hist2d, xc, wc, bc2d, ec, hp, w1rl, b1rl, w1rr, w1fl,
                 b1fl, w1fr, w2rl, b2rl, w2rr, w2fl, b2fl, w2fr):
    ae0 = hist2d[0 * NP:1 * NP]
    ao0 = hist2d[1 * NP:2 * NP]
    ae1 = hist2d[2 * NP:3 * NP]
    ao1 = hist2d[3 * NP:4 * NP]
    return pl.pallas_call(
        _dense_body,
        out_shape=jax.ShapeDtypeStruct((2 * SROWS, 128), jnp.float32),
    )(ae0, ao0, ae1, ao1, xc, wc, bc2d, ec, hp, w1rl, b1rl,
      w1rr, w1fl, b1fl, w1fr, w2rl, b2rl, w2rr, w2fl, b2fl, w2fr)


def _product_encoder(x_product, w, b2d, emb):
    return pl.pallas_call(
        _enc_body,
        grid=(NP // ENC_ROWS,),
        in_specs=[
            pl.BlockSpec((ENC_ROWS, NP), lambda i: (i, 0)),
            pl.BlockSpec((NP, HID), lambda i: (0, 0)),
            pl.BlockSpec((1, HID), lambda i: (0, 0)),
            pl.BlockSpec((ENC_ROWS, HID), lambda i: (i, 0)),
        ],
        out_specs=pl.BlockSpec((ENC_ROWS, HID), lambda i: (i, 0)),
        out_shape=jax.ShapeDtypeStruct((NP, HID), jnp.float32),
    )(x_product, w, b2d, emb)


def _dense_stack(hist2d, xc, wc, bc2d, ec, hp, w1rl, b1rl, w1rr, w1fl,
                 b1fl, w1fr, w2rl, b2rl, w2rr, w2fl, b2fl, w2fr):
    ae0 = hist2d[0 * NP:1 * NP]
    ao0 = hist2d[1 * NP:2 * NP]
    ae1 = hist2d[2 * NP:3 * NP]
    ao1 = hist2d[3 * NP:4 * NP]
    return pl.pallas_call(
        _dense_body,
        out_shape=jax.ShapeDtypeStruct((2 * SROWS, 128), jnp.float32),
    )(ae0, ao0, ae1, ao1, xc, wc, bc2d, ec, hp, w1rl, b1rl,
      w1rr, w1fl, b1fl, w1fr, w2rl, b2rl, w2rr, w2fl, b2fl, w2fr)


def kernel(x_country, x_product, W_country_lin, b_country_lin, W_product_lin,
           b_product_lin, emb_country, emb_product, W1f_l, b1f_l, W1f_r,
           W1r_l, b1r_l, W1r_r, W2f_l, b2f_l, W2f_r, W2r_l, b2r_l, W2r_r,
           node_id_country, node_id_product, edge_src, edge_dst, label_src,
           label_dst):
    del node_id_country, node_id_product

    zeros = jnp.zeros((NPAD,), jnp.float32)
    hist = _edge_histogram()(edge_src, edge_dst, zeros)
    hist2d = hist.reshape(2 * NPAD // 128, 128)

    h_p = _product_encoder(
        x_product, W_product_lin, b_product_lin.reshape(1, HID), emb_product)

    s = _dense_stack(
        hist2d, x_country, W_country_lin, b_country_lin.reshape(1, HID),
        emb_country, h_p,
        W1r_l, b1r_l.reshape(1, HID), W1r_r,
        W1f_l, b1f_l.reshape(1, HID), W1f_r,
        W2r_l, b2r_l.reshape(1, HID), W2r_r,
        W2f_l, b2f_l.reshape(1, HID), W2f_r)

    pad = LPAD - L
    fill = jnp.arange(pad, dtype=jnp.int32)
    ls = jnp.concatenate([label_src, fill % NC])
    ld = jnp.concatenate([label_dst, fill % NP])
    out_pad = _label_gather()(ls, ld, s.reshape(-1))
    return out_pad.reshape(32, GROWS * 128)[:, :LPW].reshape(-1)[:L]


# speedup vs baseline: 1.0963x; 1.0963x over previous
import functools

import jax
import jax.numpy as jnp
from jax import lax
from jax.experimental import pallas as pl
from jax.experimental.pallas import tpu as pltpu
from jax.experimental.pallas import tpu_sc as plsc

HID = 128
NC = 250
NP = 5000
E = 512000
L = 200000

STRIPE = 80000
NPAD = 16 * STRIPE
ZCHUNK = 8000
EPW = E // 32
LPW = 6272
LPAD = 32 * LPW
GR = LPW // 128
AHALF = NP * 128
SCOLS = 5120
ENC_ROWS = 1000
SROWS = 5120
SHALF = SROWS * 128

def _sc_mesh():
    return plsc.VectorSubcoreMesh(
        core_axis_name="c", subcore_axis_name="s", num_cores=2,
        num_subcores=16)


def _hist_body(esrc, edst, zeros, out, acc, src_v, dst_v, idx_v, ones_v, sem):
    c = lax.axis_index("c")
    s = lax.axis_index("s")
    stripe = pl.ds(s * STRIPE, STRIPE)

    one16 = jnp.ones((16,), jnp.float32)

    def fill_ones(i, _):
        ones_v[pl.ds(i * 16, 16)] = one16
        return 0
    lax.fori_loop(0, 128 // 16, fill_ones, 0)

    pltpu.sync_copy(zeros.at[stripe], acc.at[stripe])

    base = (c * 16 + s) * EPW
    pltpu.sync_copy(esrc.at[pl.ds(base, EPW)], src_v)
    pltpu.sync_copy(edst.at[pl.ds(base, EPW)], dst_v)
    plsc.subcore_barrier()

    def row(j, _):
        for k in range(8):
            off = j * 128 + k * 16
            sv = src_v[pl.ds(off, 16)]
            dv = dst_v[pl.ds(off, 16)]
            hbase = (sv >> 7) * (AHALF - 128)
            idx_v[j, pl.ds(k * 16, 16)] = hbase + dv * 128 + sv
        pltpu.async_copy(ones_v, acc.at[idx_v.at[j]], sem, add=True)
        return 0
    lax.fori_loop(0, EPW // 128, row, 0)

    def drain(j, _):
        pltpu.make_async_copy(ones_v, acc.at[idx_v.at[0]], sem).wait()
        return 0
    lax.fori_loop(0, EPW // 128, drain, 0)
    plsc.subcore_barrier()

    pltpu.sync_copy(acc.at[stripe],
                    out.at[pl.ds(c * NPAD + s * STRIPE, STRIPE)])


@functools.cache
def _edge_histogram():
    return pl.kernel(
        _hist_body,
        out_type=jax.ShapeDtypeStruct((2 * NPAD,), jnp.float32),
        mesh=_sc_mesh(),
        scratch_types=[
            pltpu.VMEM_SHARED((NPAD,), jnp.float32),
            pltpu.VMEM((EPW,), jnp.int32),
            pltpu.VMEM((EPW,), jnp.int32),
            pltpu.VMEM((EPW // 128, 128), jnp.int32),
            pltpu.VMEM((128,), jnp.float32),
            pltpu.SemaphoreType.DMA,
        ],
    )


GROWS = 56


def _gather_body(ls, ld, stab, out, sv_ref, dv_ref, idx_v, out_v, sem):
    c = lax.axis_index("c")
    s = lax.axis_index("s")
    w = c * 16 + s
    base = w * LPW
    pltpu.sync_copy(ls.at[pl.ds(base, LPW)], sv_ref)
    pltpu.sync_copy(ld.at[pl.ds(base, LPW)], dv_ref)

    def row(j, _):
        for k in range(8):
            off = j * 128 + k * 16
            a = sv_ref[pl.ds(off, 16)]
            b = dv_ref[pl.ds(off, 16)]
            gbase = (a >> 7) * (SHALF - 128)
            idx_v[j, pl.ds(k * 16, 16)] = gbase + b * 128 + a
        pltpu.async_copy(stab.at[idx_v.at[j]], out_v.at[j], sem)
        return 0
    lax.fori_loop(0, GR, row, 0)

    def drain(j, _):
        pltpu.make_async_copy(stab.at[idx_v.at[0]], out_v.at[0], sem).wait()
        return 0
    lax.fori_loop(0, GR, drain, 0)
    pltpu.sync_copy(out_v, out.at[pl.ds(w * GROWS, GROWS)])


@functools.cache
def _label_gather():
    return pl.kernel(
        _gather_body,
        out_type=jax.ShapeDtypeStruct((32 * GROWS, 128), jnp.float32),
        mesh=_sc_mesh(),
        scratch_types=[
            pltpu.VMEM((LPW,), jnp.int32),
            pltpu.VMEM((LPW,), jnp.int32),
            pltpu.VMEM((GR, 128), jnp.int32),
            pltpu.VMEM((GROWS, 128), jnp.float32),
            pltpu.SemaphoreType.DMA,
        ],
    )


def _enc_body(x_ref, w_ref, b_ref, e_ref, o_ref):
    o_ref[...] = (
        jnp.dot(x_ref[...], w_ref[...], preferred_element_type=jnp.float32)
        + b_ref[...] + e_ref[...])


def _dense_body(ae0_ref, ao0_ref, ae1_ref, ao1_ref, xc_ref, wc_ref, bc_ref,
                ec_ref, hp_ref,
                w1rl_ref, b1rl_ref, w1rr_ref, w1fl_ref, b1fl_ref, w1fr_ref,
                w2rl_ref, b2rl_ref, w2rr_ref, w2fl_ref, b2fl_ref, w2fr_ref,
                s_ref):
    f32 = jnp.float32

    def dot(p, q):
        return lax.dot_general(p, q, (((1,), (0,)), ((), ())),
                               preferred_element_type=f32)

    def dot0(p, q):
        return lax.dot_general(p, q, (((0,), (0,)), ((), ())),
                               preferred_element_type=f32)

    ae = ae0_ref[...] + ae1_ref[...]
    ao = ao0_ref[...] + ao1_ref[...]
    rinv = 1.0 / jnp.maximum(
        jnp.sum(ae, axis=1, keepdims=True) + jnp.sum(ao, axis=1, keepdims=True),
        1.0)
    an_e = ae * rinv
    an_o = ao * rinv
    atn_e = ae / jnp.maximum(jnp.sum(ae, axis=0, keepdims=True), 1.0)
    atn_o = ao / jnp.maximum(jnp.sum(ao, axis=0, keepdims=True), 1.0)

    h_c = dot(xc_ref[...], wc_ref[...]) + bc_ref[...] + ec_ref[...]
    h_p = hp_ref[...]
    zpad = jnp.zeros((256 - NC, HID), f32)

    def mean_p(xc256):
        return dot(an_e, xc256[:128]) + dot(an_o, xc256[128:])

    def mean_c(xp):
        return jnp.concatenate([dot0(atn_e, xp), dot0(atn_o, xp)], axis=0)[:NC]

    hcp = jnp.concatenate([h_c, zpad], axis=0)
    h_c1 = jnp.maximum(
        dot(mean_c(h_p), w1rl_ref[...]) + b1rl_ref[...]
        + dot(h_c, w1rr_ref[...]), 0.0)
    h_p1 = jnp.maximum(
        dot(mean_p(hcp), w1fl_ref[...]) + b1fl_ref[...]
        + dot(h_p, w1fr_ref[...]), 0.0)

    hc1p = jnp.concatenate([h_c1, zpad], axis=0)
    h_c2 = (dot(mean_c(h_p1), w2rl_ref[...]) + b2rl_ref[...]
            + dot(h_c1, w2rr_ref[...]))
    h_p2 = (dot(mean_p(hc1p), w2fl_ref[...]) + b2fl_ref[...]
            + dot(h_p1, w2fr_ref[...]))

    hc2p = jnp.concatenate([h_c2, zpad], axis=0)
    oe = lax.dot_general(h_p2, hc2p[:128], (((1,), (1,)), ((), ())),
                         preferred_element_type=f32)
    oo = lax.dot_general(h_p2, hc2p[128:], (((1,), (1,)), ((), ())),
                         preferred_element_type=f32)
    spad = jnp.zeros((SROWS - NP, 128), f32)
    s_ref[...] = jnp.concatenate([oe, spad, oo, spad], axis=0)


def _dense_stack(hist2d, xc, wc, bc2d, ec, hp, w1rl, b1rl, w1rr, w1fl,
                 b1fl, w1fr, w2rl, b2rl, w2rr, w2fl, b2fl, w2fr):
    full = lambda arr: pl.BlockSpec(arr.shape, lambda i: (0,) * arr.ndim)
    in_specs = [
        pl.BlockSpec((NP, 128), lambda i: (0, 0)),
        pl.BlockSpec((NP, 128), lambda i: (1, 0)),
        pl.BlockSpec((NP, 128), lambda i: (2, 0)),
        pl.BlockSpec((NP, 128), lambda i: (3, 0)),
    ] + [full(a) for a in (xc, wc, bc2d, ec, hp, w1rl, b1rl, w1rr, w1fl,
                           b1fl, w1fr, w2rl, b2rl, w2rr, w2fl, b2fl, w2fr)]
    return pl.pallas_call(
        _dense_body,
        grid=(1,),
        in_specs=in_specs,
        out_specs=pl.BlockSpec((2 * SROWS, 128), lambda i: (0, 0)),
        out_shape=jax.ShapeDtypeStruct((2 * SROWS, 128), jnp.float32),
    )(hist2d, hist2d, hist2d, hist2d, xc, wc, bc2d, ec, hp, w1rl, b1rl,
      w1rr, w1fl, b1fl, w1fr, w2rl, b2rl, w2rr, w2fl, b2fl, w2fr)


def _product_encoder(x_product, w, b2d, emb):
    return pl.pallas_call(
        _enc_body,
        grid=(NP // ENC_ROWS,),
        in_specs=[
            pl.BlockSpec((ENC_ROWS, NP), lambda i: (i, 0)),
            pl.BlockSpec((NP, HID), lambda i: (0, 0)),
            pl.BlockSpec((1, HID), lambda i: (0, 0)),
            pl.BlockSpec((ENC_ROWS, HID), lambda i: (i, 0)),
        ],
        out_specs=pl.BlockSpec((ENC_ROWS, HID), lambda i: (i, 0)),
        out_shape=jax.ShapeDtypeStruct((NP, HID), jnp.float32),
    )(x_product, w, b2d, emb)


def _dense_stack(hist2d, xc, wc, bc2d, ec, hp, w1rl, b1rl, w1rr, w1fl,
                 b1fl, w1fr, w2rl, b2rl, w2rr, w2fl, b2fl, w2fr):
    full = lambda arr: pl.BlockSpec(arr.shape, lambda i: (0,) * arr.ndim)
    in_specs = [
        pl.BlockSpec((NP, 128), lambda i: (0, 0)),
        pl.BlockSpec((NP, 128), lambda i: (1, 0)),
        pl.BlockSpec((NP, 128), lambda i: (2, 0)),
        pl.BlockSpec((NP, 128), lambda i: (3, 0)),
    ] + [full(a) for a in (xc, wc, bc2d, ec, hp, w1rl, b1rl, w1rr, w1fl,
                           b1fl, w1fr, w2rl, b2rl, w2rr, w2fl, b2fl, w2fr)]
    return pl.pallas_call(
        _dense_body,
        grid=(1,),
        in_specs=in_specs,
        out_specs=pl.BlockSpec((2 * SROWS, 128), lambda i: (0, 0)),
        out_shape=jax.ShapeDtypeStruct((2 * SROWS, 128), jnp.float32),
    )(hist2d, hist2d, hist2d, hist2d, xc, wc, bc2d, ec, hp, w1rl, b1rl,
      w1rr, w1fl, b1fl, w1fr, w2rl, b2rl, w2rr, w2fl, b2fl, w2fr)


def kernel(x_country, x_product, W_country_lin, b_country_lin, W_product_lin,
           b_product_lin, emb_country, emb_product, W1f_l, b1f_l, W1f_r,
           W1r_l, b1r_l, W1r_r, W2f_l, b2f_l, W2f_r, W2r_l, b2r_l, W2r_r,
           node_id_country, node_id_product, edge_src, edge_dst, label_src,
           label_dst):
    del node_id_country, node_id_product

    zeros = jnp.zeros((NPAD,), jnp.float32)
    hist = _edge_histogram()(edge_src, edge_dst, zeros)
    hist2d = hist.reshape(2 * NPAD // 128, 128)

    h_p = _product_encoder(
        x_product, W_product_lin, b_product_lin.reshape(1, HID), emb_product)

    s = _dense_stack(
        hist2d, x_country, W_country_lin, b_country_lin.reshape(1, HID),
        emb_country, h_p,
        W1r_l, b1r_l.reshape(1, HID), W1r_r,
        W1f_l, b1f_l.reshape(1, HID), W1f_r,
        W2r_l, b2r_l.reshape(1, HID), W2r_r,
        W2f_l, b2f_l.reshape(1, HID), W2f_r)

    pad = LPAD - L
    fill = jnp.arange(pad, dtype=jnp.int32)
    ls = jnp.concatenate([label_src, fill % NC])
    ld = jnp.concatenate([label_dst, fill % NP])
    out_pad = _label_gather()(ls, ld, s.reshape(-1))
    return out_pad.reshape(32, GROWS * 128)[:, :LPW].reshape(-1)[:L]


# speedup vs baseline: 1.1445x; 1.0439x over previous
import functools

import jax
import jax.numpy as jnp
from jax import lax
from jax.experimental import pallas as pl
from jax.experimental.pallas import tpu as pltpu
from jax.experimental.pallas import tpu_sc as plsc

HID = 128
NC = 250
NP = 5000
E = 512000
L = 200000

STRIPE = 80000
NPAD = 16 * STRIPE
ZCHUNK = 2000
EPW = E // 32
LPW = 6272
LPAD = 32 * LPW
GR = LPW // 128
AHALF = NP * 128
SCOLS = 5120
ENC_ROWS = 1000
SROWS = 5120
SHALF = SROWS * 128

def _sc_mesh():
    return plsc.VectorSubcoreMesh(
        core_axis_name="c", subcore_axis_name="s", num_cores=2,
        num_subcores=16)


def _hist_body(esrc, edst, out, acc, src_v, dst_v, idx_v, ones_v, zbuf, sem):
    c = lax.axis_index("c")
    s = lax.axis_index("s")
    stripe = pl.ds(s * STRIPE, STRIPE)

    one16 = jnp.ones((16,), jnp.float32)
    zero16 = jnp.zeros((16,), jnp.float32)

    def fill_ones(i, _):
        ones_v[pl.ds(i * 16, 16)] = one16
        return 0
    lax.fori_loop(0, 128 // 16, fill_ones, 0)

    def fill_zb(i, _):
        zbuf[pl.ds(i * 16, 16)] = zero16
        return 0
    lax.fori_loop(0, ZCHUNK // 16, fill_zb, 0)

    def fire_zero(i, _):
        pltpu.async_copy(zbuf, acc.at[pl.ds(s * STRIPE + i * ZCHUNK, ZCHUNK)],
                         sem)
        return 0
    lax.fori_loop(0, STRIPE // ZCHUNK, fire_zero, 0)

    base = (c * 16 + s) * EPW
    pltpu.sync_copy(esrc.at[pl.ds(base, EPW)], src_v)
    pltpu.sync_copy(edst.at[pl.ds(base, EPW)], dst_v)

    def drain_zero(i, _):
        pltpu.make_async_copy(
            zbuf, acc.at[pl.ds(s * STRIPE, ZCHUNK)], sem).wait()
        return 0
    lax.fori_loop(0, STRIPE // ZCHUNK, drain_zero, 0)
    plsc.subcore_barrier()

    def row(j, _):
        for k in range(8):
            off = j * 128 + k * 16
            sv = src_v[pl.ds(off, 16)]
            dv = dst_v[pl.ds(off, 16)]
            hbase = (sv >> 7) * (AHALF - 128)
            idx_v[j, pl.ds(k * 16, 16)] = hbase + dv * 128 + sv
        pltpu.async_copy(ones_v, acc.at[idx_v.at[j]], sem, add=True)
        return 0
    lax.fori_loop(0, EPW // 128, row, 0)

    def drain(j, _):
        pltpu.make_async_copy(ones_v, acc.at[idx_v.at[0]], sem).wait()
        return 0
    lax.fori_loop(0, EPW // 128, drain, 0)
    plsc.subcore_barrier()

    pltpu.sync_copy(acc.at[stripe],
                    out.at[pl.ds(c * NPAD + s * STRIPE, STRIPE)])


@functools.cache
def _edge_histogram():
    return pl.kernel(
        _hist_body,
        out_type=jax.ShapeDtypeStruct((2 * NPAD,), jnp.float32),
        mesh=_sc_mesh(),
        scratch_types=[
            pltpu.VMEM_SHARED((NPAD,), jnp.float32),
            pltpu.VMEM((EPW,), jnp.int32),
            pltpu.VMEM((EPW,), jnp.int32),
            pltpu.VMEM((EPW // 128, 128), jnp.int32),
            pltpu.VMEM((128,), jnp.float32),
            pltpu.VMEM((ZCHUNK,), jnp.float32),
            pltpu.SemaphoreType.DMA,
        ],
    )


GROWS = 56


def _gather_body(ls, ld, stab, out, sv_ref, dv_ref, idx_v, out_v, sem):
    c = lax.axis_index("c")
    s = lax.axis_index("s")
    w = c * 16 + s
    base = w * LPW
    pltpu.sync_copy(ls.at[pl.ds(base, LPW)], sv_ref)
    pltpu.sync_copy(ld.at[pl.ds(base, LPW)], dv_ref)

    def row(j, _):
        for k in range(8):
            off = j * 128 + k * 16
            a = sv_ref[pl.ds(off, 16)]
            b = dv_ref[pl.ds(off, 16)]
            gbase = (a >> 7) * (SHALF - 128)
            idx_v[j, pl.ds(k * 16, 16)] = gbase + b * 128 + a
        pltpu.async_copy(stab.at[idx_v.at[j]], out_v.at[j], sem)
        return 0
    lax.fori_loop(0, GR, row, 0)

    def drain(j, _):
        pltpu.make_async_copy(stab.at[idx_v.at[0]], out_v.at[0], sem).wait()
        return 0
    lax.fori_loop(0, GR, drain, 0)
    pltpu.sync_copy(out_v, out.at[pl.ds(w * GROWS, GROWS)])


@functools.cache
def _label_gather():
    return pl.kernel(
        _gather_body,
        out_type=jax.ShapeDtypeStruct((32 * GROWS, 128), jnp.float32),
        mesh=_sc_mesh(),
        scratch_types=[
            pltpu.VMEM((LPW,), jnp.int32),
            pltpu.VMEM((LPW,), jnp.int32),
            pltpu.VMEM((GR, 128), jnp.int32),
            pltpu.VMEM((GROWS, 128), jnp.float32),
            pltpu.SemaphoreType.DMA,
        ],
    )


def _enc_body(x_ref, w_ref, b_ref, e_ref, o_ref):
    o_ref[...] = (
        jnp.dot(x_ref[...], w_ref[...], preferred_element_type=jnp.float32)
        + b_ref[...] + e_ref[...])


def _dense_body(ae0_ref, ao0_ref, ae1_ref, ao1_ref, xc_ref, wc_ref, bc_ref,
                ec_ref, hp_ref,
                w1rl_ref, b1rl_ref, w1rr_ref, w1fl_ref, b1fl_ref, w1fr_ref,
                w2rl_ref, b2rl_ref, w2rr_ref, w2fl_ref, b2fl_ref, w2fr_ref,
                s_ref):
    f32 = jnp.float32

    def dot(p, q):
        return lax.dot_general(p, q, (((1,), (0,)), ((), ())),
                               preferred_element_type=f32)

    def dot0(p, q):
        return lax.dot_general(p, q, (((0,), (0,)), ((), ())),
                               preferred_element_type=f32)

    ae = ae0_ref[...] + ae1_ref[...]
    ao = ao0_ref[...] + ao1_ref[...]
    rinv = 1.0 / jnp.maximum(
        jnp.sum(ae, axis=1, keepdims=True) + jnp.sum(ao, axis=1, keepdims=True),
        1.0)
    an_e = ae * rinv
    an_o = ao * rinv
    atn_e = ae / jnp.maximum(jnp.sum(ae, axis=0, keepdims=True), 1.0)
    atn_o = ao / jnp.maximum(jnp.sum(ao, axis=0, keepdims=True), 1.0)

    h_c = dot(xc_ref[...], wc_ref[...]) + bc_ref[...] + ec_ref[...]
    h_p = hp_ref[...]
    zpad = jnp.zeros((256 - NC, HID), f32)

    def mean_p(xc256):
        return dot(an_e, xc256[:128]) + dot(an_o, xc256[128:])

    def mean_c(xp):
        return jnp.concatenate([dot0(atn_e, xp), dot0(atn_o, xp)], axis=0)[:NC]

    hcp = jnp.concatenate([h_c, zpad], axis=0)
    h_c1 = jnp.maximum(
        dot(mean_c(h_p), w1rl_ref[...]) + b1rl_ref[...]
        + dot(h_c, w1rr_ref[...]), 0.0)
    h_p1 = jnp.maximum(
        dot(mean_p(hcp), w1fl_ref[...]) + b1fl_ref[...]
        + dot(h_p, w1fr_ref[...]), 0.0)

    hc1p = jnp.concatenate([h_c1, zpad], axis=0)
    h_c2 = (dot(mean_c(h_p1), w2rl_ref[...]) + b2rl_ref[...]
            + dot(h_c1, w2rr_ref[...]))
    h_p2 = (dot(mean_p(hc1p), w2fl_ref[...]) + b2fl_ref[...]
            + dot(h_p1, w2fr_ref[...]))

    hc2p = jnp.concatenate([h_c2, zpad], axis=0)
    oe = lax.dot_general(h_p2, hc2p[:128], (((1,), (1,)), ((), ())),
                         preferred_element_type=f32)
    oo = lax.dot_general(h_p2, hc2p[128:], (((1,), (1,)), ((), ())),
                         preferred_element_type=f32)
    spad = jnp.zeros((SROWS - NP, 128), f32)
    s_ref[...] = jnp.concatenate([oe, spad, oo, spad], axis=0)


def _dense_stack(hist2d, xc, wc, bc2d, ec, hp, w1rl, b1rl, w1rr, w1fl,
                 b1fl, w1fr, w2rl, b2rl, w2rr, w2fl, b2fl, w2fr):
    full = lambda arr: pl.BlockSpec(arr.shape, lambda i: (0,) * arr.ndim)
    in_specs = [
        pl.BlockSpec((NP, 128), lambda i: (0, 0)),
        pl.BlockSpec((NP, 128), lambda i: (1, 0)),
        pl.BlockSpec((NP, 128), lambda i: (2, 0)),
        pl.BlockSpec((NP, 128), lambda i: (3, 0)),
    ] + [full(a) for a in (xc, wc, bc2d, ec, hp, w1rl, b1rl, w1rr, w1fl,
                           b1fl, w1fr, w2rl, b2rl, w2rr, w2fl, b2fl, w2fr)]
    return pl.pallas_call(
        _dense_body,
        grid=(1,),
        in_specs=in_specs,
        out_specs=pl.BlockSpec((2 * SROWS, 128), lambda i: (0, 0)),
        out_shape=jax.ShapeDtypeStruct((2 * SROWS, 128), jnp.float32),
    )(hist2d, hist2d, hist2d, hist2d, xc, wc, bc2d, ec, hp, w1rl, b1rl,
      w1rr, w1fl, b1fl, w1fr, w2rl, b2rl, w2rr, w2fl, b2fl, w2fr)


def _product_encoder(x_product, w, b2d, emb):
    return pl.pallas_call(
        _enc_body,
        grid=(NP // ENC_ROWS,),
        in_specs=[
            pl.BlockSpec((ENC_ROWS, NP), lambda i: (i, 0)),
            pl.BlockSpec((NP, HID), lambda i: (0, 0)),
            pl.BlockSpec((1, HID), lambda i: (0, 0)),
            pl.BlockSpec((ENC_ROWS, HID), lambda i: (i, 0)),
        ],
        out_specs=pl.BlockSpec((ENC_ROWS, HID), lambda i: (i, 0)),
        out_shape=jax.ShapeDtypeStruct((NP, HID), jnp.float32),
    )(x_product, w, b2d, emb)


def _dense_stack(hist2d, xc, wc, bc2d, ec, hp, w1rl, b1rl, w1rr, w1fl,
                 b1fl, w1fr, w2rl, b2rl, w2rr, w2fl, b2fl, w2fr):
    full = lambda arr: pl.BlockSpec(arr.shape, lambda i: (0,) * arr.ndim)
    in_specs = [
        pl.BlockSpec((NP, 128), lambda i: (0, 0)),
        pl.BlockSpec((NP, 128), lambda i: (1, 0)),
        pl.BlockSpec((NP, 128), lambda i: (2, 0)),
        pl.BlockSpec((NP, 128), lambda i: (3, 0)),
    ] + [full(a) for a in (xc, wc, bc2d, ec, hp, w1rl, b1rl, w1rr, w1fl,
                           b1fl, w1fr, w2rl, b2rl, w2rr, w2fl, b2fl, w2fr)]
    return pl.pallas_call(
        _dense_body,
        grid=(1,),
        in_specs=in_specs,
        out_specs=pl.BlockSpec((2 * SROWS, 128), lambda i: (0, 0)),
        out_shape=jax.ShapeDtypeStruct((2 * SROWS, 128), jnp.float32),
    )(hist2d, hist2d, hist2d, hist2d, xc, wc, bc2d, ec, hp, w1rl, b1rl,
      w1rr, w1fl, b1fl, w1fr, w2rl, b2rl, w2rr, w2fl, b2fl, w2fr)


def kernel(x_country, x_product, W_country_lin, b_country_lin, W_product_lin,
           b_product_lin, emb_country, emb_product, W1f_l, b1f_l, W1f_r,
           W1r_l, b1r_l, W1r_r, W2f_l, b2f_l, W2f_r, W2r_l, b2r_l, W2r_r,
           node_id_country, node_id_product, edge_src, edge_dst, label_src,
           label_dst):
    del node_id_country, node_id_product

    hist = _edge_histogram()(edge_src, edge_dst)
    hist2d = hist.reshape(2 * NPAD // 128, 128)

    h_p = _product_encoder(
        x_product, W_product_lin, b_product_lin.reshape(1, HID), emb_product)

    s = _dense_stack(
        hist2d, x_country, W_country_lin, b_country_lin.reshape(1, HID),
        emb_country, h_p,
        W1r_l, b1r_l.reshape(1, HID), W1r_r,
        W1f_l, b1f_l.reshape(1, HID), W1f_r,
        W2r_l, b2r_l.reshape(1, HID), W2r_r,
        W2f_l, b2f_l.reshape(1, HID), W2f_r)

    pad = LPAD - L
    fill = jnp.arange(pad, dtype=jnp.int32)
    ls = jnp.concatenate([label_src, fill % NC])
    ld = jnp.concatenate([label_dst, fill % NP])
    out_pad = _label_gather()(ls, ld, s.reshape(-1))
    return out_pad.reshape(32, GROWS * 128)[:, :LPW].reshape(-1)[:L]
